# baseline (speedup 1.0000x reference)
import functools

import jax
import jax.numpy as jnp
from jax import lax
from jax.experimental import pallas as pl
from jax.experimental.pallas import tpu as pltpu
from jax.experimental.pallas import tpu_sc as plsc

N = 10000
E = 320000
D = 128

NW = 32
RANGE = 320
NP = NW * RANGE
ACC_R = RANGE + 1
G = 128
CH = 6400
NCHUNK = E // CH
UNROLL = 8
NBLK = CH // 16 // UNROLL

TBLK = 400
TGRID = N // TBLK


def _rowmat(x, w):
    return lax.dot_general(x, w, (((1,), (1,)), ((), ())),
                           preferred_element_type=jnp.float32)


def _tc1_body(x_ref, wp_ref, bp_ref, ws_ref, hp_ref, xs_ref):
    xb = x_ref[...]
    hp_ref[...] = jnp.maximum(_rowmat(xb, wp_ref[...]) + bp_ref[...], 0.0)
    xs_ref[...] = _rowmat(xb, ws_ref[...])


def _tc2_body(xs_ref, agg_ref, wn_ref, b_ref, wp_ref, bp_ref, ws_ref,
              hp_ref, hs_ref):
    h = jnp.tanh(xs_ref[...] + _rowmat(agg_ref[...], wn_ref[...]) + b_ref[...])
    hp_ref[...] = jnp.maximum(_rowmat(h, wp_ref[...]) + bp_ref[...], 0.0)
    hs_ref[...] = _rowmat(h, ws_ref[...])


def _tc3_body(hs_ref, agg_ref, wn_ref, b_ref, out_ref):
    out_ref[...] = hs_ref[...] + _rowmat(agg_ref[...], wn_ref[...]) + b_ref[...]


_row_spec = pl.BlockSpec((TBLK, D), lambda i: (i, 0))
_w_spec = pl.BlockSpec((D, D), lambda i: (0, 0))
_b_spec = pl.BlockSpec((1, D), lambda i: (0, 0))
_n_out = jax.ShapeDtypeStruct((N, D), jnp.float32)

_mm1 = pl.pallas_call(
    _tc1_body, grid=(TGRID,),
    in_specs=[_row_spec, _w_spec, _b_spec, _w_spec],
    out_specs=[_row_spec, _row_spec],
    out_shape=[_n_out, _n_out])

_mm2 = pl.pallas_call(
    _tc2_body, grid=(TGRID,),
    in_specs=[_row_spec, _row_spec, _w_spec, _b_spec, _w_spec, _b_spec,
              _w_spec],
    out_specs=[_row_spec, _row_spec],
    out_shape=[_n_out, _n_out])

_mm3 = pl.pallas_call(
    _tc3_body, grid=(TGRID,),
    in_specs=[_row_spec, _row_spec, _w_spec, _b_spec],
    out_specs=_row_spec,
    out_shape=_n_out)


_mesh = plsc.VectorSubcoreMesh(core_axis_name="c", subcore_axis_name="s")


@functools.partial(
    pl.kernel, mesh=_mesh,
    out_type=jax.ShapeDtypeStruct((NP, D), jnp.float32),
    compiler_params=pltpu.CompilerParams(needs_layout_passes=False),
    scratch_types=[
        pltpu.VMEM((2 * CH,), jnp.int32),
        pltpu.VMEM((2 * CH,), jnp.int32),
        pltpu.VMEM((256,), jnp.int32),
        pltpu.VMEM((2 * G,), jnp.int32),
        pltpu.VMEM((2 * G,), jnp.int32),
        pltpu.VMEM((2, G, D), jnp.float32),
        pltpu.VMEM((ACC_R, D), jnp.float32),
        pltpu.SemaphoreType.DMA((2,)),
        pltpu.SemaphoreType.DMA((2,)),
        pltpu.SemaphoreType.DMA((2,)),
    ])
def _segmax(hp_hbm, src_hbm, dst_hbm, out_hbm,
            dst_buf, src_buf, pend, idx2, dlq, rows2, acc,
            dsem, ssem, gsem):
    wid = lax.axis_index("s") * 2 + lax.axis_index("c")
    lo = wid * RANGE
    zero16f = jnp.zeros((16,), jnp.float32)
    zero16 = jnp.zeros((16,), jnp.int32)
    one16 = zero16 + 1
    eight16 = zero16 + 8
    fifteen16 = zero16 + 15
    lane16 = jnp.arange(16, dtype=jnp.int32)
    lo_vec = zero16 + lo
    hi_vec = lo_vec + RANGE
    stale_vec = (lo_vec << 9) | RANGE

    def _zero_row(i, carry):
        for j in range(D // 16):
            acc[i, pl.ds(j * 16, 16)] = zero16f
        return carry

    lax.fori_loop(0, ACC_R, _zero_row, 0)

    for k in range(256 // 16):
        pend[pl.ds(k * 16, 16)] = stale_vec

    def _proc(q):
        def _upd(blk, carry):
            base = blk * 16
            dlv = dlq[pl.ds(q * G + base, 16)]
            for k in range(16):
                dl = dlv[k]
                for j in range(D // 16):
                    acc[dl, pl.ds(j * 16, 16)] = jnp.maximum(
                        acc[dl, pl.ds(j * 16, 16)],
                        rows2[q, base + k, pl.ds(j * 16, 16)])
            return carry

        lax.fori_loop(0, G // 16, _upd, 0)

    def _fire(args):
        pend_vec, drainp, fcnt = args
        p = lax.rem(fcnt, 2)
        for r in range(G // 16):
            v = pend[pl.ds(p * G + r * 16, 16)]
            idx2[pl.ds(p * G + r * 16, 16)] = jnp.right_shift(v, 9)
            dlq[pl.ds(p * G + r * 16, 16)] = v & 511
        pltpu.async_copy(hp_hbm.at[idx2.at[pl.ds(p * G, G)]], rows2.at[p],
                         gsem.at[p])
        @pl.when(fcnt > 0)
        def _():
            q = 1 - p
            pltpu.make_async_copy(hp_hbm.at[idx2.at[pl.ds(q * G, G)]],
                                  rows2.at[q], gsem.at[q]).wait()
            _proc(q)
        return jnp.maximum(pend_vec - eight16, zero16), drainp + eight16, \
            fcnt + 1

    def _group(cb, g, carry):
        pend_vec, drainp = carry
        d = dst_buf[pl.ds(cb * CH + g * 16, 16)]
        s = src_buf[pl.ds(cb * CH + g * 16, 16)]
        m = (d >= lo_vec) & (d < hi_vec)
        tgt = (((drainp + pend_vec) & fifteen16) << 4) + lane16
        val = (s << 9) | (d - lo_vec)
        plsc.store_scatter(pend, [tgt], val, mask=m)
        return pend_vec + jnp.where(m, one16, zero16), drainp

    def _make_block(cb):
        def _block(b, carry):
            pend_vec, drainp, fcnt = carry
            for k in range(UNROLL):
                pend_vec, drainp = _group(cb, b * UNROLL + k,
                                          (pend_vec, drainp))
            any8 = plsc.all_reduce_population_count(pend_vec >= eight16)
            return lax.cond(any8[0] > 0, _fire,
                            lambda a: a, (pend_vec, drainp, fcnt))
        return _block

    def _start_chunk(c):
        cb = lax.rem(c, 2)
        pltpu.async_copy(dst_hbm.at[pl.ds(c * CH, CH)],
                         dst_buf.at[pl.ds(cb * CH, CH)], dsem.at[cb])
        pltpu.async_copy(src_hbm.at[pl.ds(c * CH, CH)],
                         src_buf.at[pl.ds(cb * CH, CH)], ssem.at[cb])

    def _chunk(c, carry):
        cb = lax.rem(c, 2)
        pltpu.make_async_copy(dst_hbm.at[pl.ds(c * CH, CH)],
                              dst_buf.at[pl.ds(cb * CH, CH)],
                              dsem.at[cb]).wait()
        pltpu.make_async_copy(src_hbm.at[pl.ds(c * CH, CH)],
                              src_buf.at[pl.ds(cb * CH, CH)],
                              ssem.at[cb]).wait()

        @pl.when(c + 1 < NCHUNK)
        def _():
            _start_chunk(c + 1)

        return lax.fori_loop(0, NBLK, _make_block(cb), carry)

    _start_chunk(0)
    carry = lax.fori_loop(0, NCHUNK, _chunk, (zero16, zero16, 0))
    carry = _fire(carry)
    carry = _fire(carry)
    _, _, fcnt = carry
    p_last = lax.rem(fcnt - 1, 2)
    pltpu.make_async_copy(hp_hbm.at[idx2.at[pl.ds(p_last * G, G)]],
                          rows2.at[p_last], gsem.at[p_last]).wait()
    _proc(p_last)
    pltpu.sync_copy(acc.at[pl.ds(0, RANGE)], out_hbm.at[pl.ds(lo, RANGE)])


def kernel(x, edge_index, Wp1, bp1, Ws1, Wn1, b1, Wp2, bp2, Ws2, Wn2, b2):
    src = edge_index[0]
    dst = edge_index[1]
    hp1, xs1 = _mm1(x, Wp1, bp1.reshape(1, D), Ws1)
    agg1 = _segmax(hp1, src, dst)
    hp2, hs2 = _mm2(xs1, agg1, Wn1, b1.reshape(1, D), Wp2, bp2.reshape(1, D),
                    Ws2)
    agg2 = _segmax(hp2, src, dst)
    out = _mm3(hs2, agg2, Wn2, b2.reshape(1, D))
    return out


# speedup vs baseline: 1.7454x; 1.7454x over previous
import functools

import jax
import jax.numpy as jnp
from jax import lax
from jax.experimental import pallas as pl
from jax.experimental.pallas import tpu as pltpu
from jax.experimental.pallas import tpu_sc as plsc

N = 10000
E = 320000
D = 128

NW = 32
RANGE = 320
NP = NW * RANGE
ACC_R = RANGE + 1
G = 128
CH = 2000
NCHUNK = E // CH
UNROLL = 5
NBLK = CH // 16 // UNROLL
PEND = 224

TBLK = 400
TGRID = N // TBLK


def _rowmat(x, w):
    return lax.dot_general(x, w, (((1,), (1,)), ((), ())),
                           preferred_element_type=jnp.float32)


def _pack_half(h):
    hb = h.astype(jnp.bfloat16)
    lo = lax.bitcast_convert_type(hb[:, :64], jnp.uint16).astype(jnp.uint32)
    hi = lax.bitcast_convert_type(hb[:, 64:], jnp.uint16).astype(jnp.uint32)
    return lax.bitcast_convert_type(lo | (hi << 16), jnp.int32)


def _unpack_half(u):
    w = lax.bitcast_convert_type(u[:, :D // 2], jnp.uint32)
    lo = lax.bitcast_convert_type((w & 0xFFFF).astype(jnp.uint16),
                                  jnp.bfloat16).astype(jnp.float32)
    hi = lax.bitcast_convert_type((w >> 16).astype(jnp.uint16),
                                  jnp.bfloat16).astype(jnp.float32)
    return jnp.concatenate([lo, hi], axis=1)


def _tc1_body(x_ref, wp_ref, bp_ref, ws_ref, hp_ref, xs_ref):
    xb = x_ref[...]
    pk = _pack_half(jnp.maximum(_rowmat(xb, wp_ref[...]) + bp_ref[...], 0.0))
    hp_ref[...] = jnp.concatenate([pk, pk], axis=1)
    xs_ref[...] = _rowmat(xb, ws_ref[...])


def _tc2_body(xs_ref, agg_ref, wn_ref, b_ref, wp_ref, bp_ref, ws_ref,
              hp_ref, hs_ref):
    agg = _unpack_half(agg_ref[...])
    h = jnp.tanh(xs_ref[...] + _rowmat(agg, wn_ref[...]) + b_ref[...])
    pk = _pack_half(jnp.maximum(_rowmat(h, wp_ref[...]) + bp_ref[...], 0.0))
    hp_ref[...] = jnp.concatenate([pk, pk], axis=1)
    hs_ref[...] = _rowmat(h, ws_ref[...])


def _tc3_body(hs_ref, agg_ref, wn_ref, b_ref, out_ref):
    agg = _unpack_half(agg_ref[...])
    out_ref[...] = hs_ref[...] + _rowmat(agg, wn_ref[...]) + b_ref[...]


_row_spec = pl.BlockSpec((TBLK, D), lambda i: (i, 0))
_pk_spec = pl.BlockSpec((TBLK, D), lambda i: (i, 0))
_w_spec = pl.BlockSpec((D, D), lambda i: (0, 0))
_b_spec = pl.BlockSpec((1, D), lambda i: (0, 0))
_n_out = jax.ShapeDtypeStruct((N, D), jnp.float32)
_pk_out = jax.ShapeDtypeStruct((N, D), jnp.int32)

_mm1 = pl.pallas_call(
    _tc1_body, grid=(TGRID,),
    in_specs=[_row_spec, _w_spec, _b_spec, _w_spec],
    out_specs=[_pk_spec, _row_spec],
    out_shape=[_pk_out, _n_out])

_mm2 = pl.pallas_call(
    _tc2_body, grid=(TGRID,),
    in_specs=[_row_spec, _pk_spec, _w_spec, _b_spec, _w_spec, _b_spec,
              _w_spec],
    out_specs=[_pk_spec, _row_spec],
    out_shape=[_pk_out, _n_out])

_mm3 = pl.pallas_call(
    _tc3_body, grid=(TGRID,),
    in_specs=[_row_spec, _pk_spec, _w_spec, _b_spec],
    out_specs=_row_spec,
    out_shape=_n_out)


_mesh = plsc.VectorSubcoreMesh(core_axis_name="c", subcore_axis_name="s")


@functools.partial(
    pl.kernel, mesh=_mesh,
    out_type=jax.ShapeDtypeStruct((NP, D), jnp.int32),
    compiler_params=pltpu.CompilerParams(needs_layout_passes=False),
    scratch_types=[
        pltpu.VMEM((2 * CH,), jnp.int32),
        pltpu.VMEM((2 * CH,), jnp.int32),
        pltpu.VMEM((PEND,), jnp.int32),
        pltpu.VMEM((PEND,), jnp.int32),
        pltpu.VMEM((2 * G,), jnp.int32),
        pltpu.VMEM((2 * G,), jnp.int32),
        pltpu.VMEM((2, G, D), jnp.int32),
        pltpu.VMEM((ACC_R, D), jnp.int32),
        pltpu.VMEM((ACC_R, D // 2), jnp.int32),
        pltpu.SemaphoreType.DMA((2,)),
        pltpu.SemaphoreType.DMA((2,)),
        pltpu.SemaphoreType.DMA((2,)),
    ])
def _segmax(hp_hbm, src_hbm, dst_hbm, out_hbm,
            dst_buf, src_buf, pend_src, pend_dl, idx2, dlq, rows2,
            acc0, acc1, dsem, ssem, gsem):
    wid = lax.axis_index("s") * 2 + lax.axis_index("c")
    lo = wid * RANGE
    zero16f = jnp.zeros((16,), jnp.float32)
    zero16 = jnp.zeros((16,), jnp.int32)
    lo_vec = zero16 + lo
    hi_vec = lo_vec + RANGE
    pad_vec = zero16 + RANGE

    def _zero_row(i, carry):
        for j in range(D // 32):
            acc0[i, pl.ds(j * 16, 16)] = zero16
            acc1[i, pl.ds(j * 16, 16)] = zero16
        return carry

    lax.fori_loop(0, ACC_R, _zero_row, 0)

    for k in range(PEND // 16):
        pend_src[pl.ds(k * 16, 16)] = lo_vec
        pend_dl[pl.ds(k * 16, 16)] = pad_vec

    def _proc(q):
        def _upd(blk, carry):
            base = blk * 16
            dlv = dlq[pl.ds(q * G + base, 16)]
            for k in range(16):
                dl = dlv[k]
                acc = acc0 if k % 2 == 0 else acc1
                for j in range(D // 32):
                    a = plsc.bitcast(acc[dl, pl.ds(j * 16, 16)],
                                     jnp.bfloat16)
                    r = plsc.bitcast(
                        rows2[q, base + k, pl.ds(j * 16, 16)], jnp.bfloat16)
                    acc[dl, pl.ds(j * 16, 16)] = plsc.bitcast(
                        jnp.maximum(a, r), jnp.int32)
            return carry

        lax.fori_loop(0, G // 16, _upd, 0)

    def _fire(args):
        pos_vec, fcnt = args
        p = lax.rem(fcnt, 2)
        for k in range(G // 16):
            idx2[pl.ds(p * G + k * 16, 16)] = pend_src[pl.ds(k * 16, 16)]
            dlq[pl.ds(p * G + k * 16, 16)] = pend_dl[pl.ds(k * 16, 16)]
        pltpu.async_copy(hp_hbm.at[idx2.at[pl.ds(p * G, G)]], rows2.at[p],
                         gsem.at[p])
        @pl.when(fcnt > 0)
        def _():
            q = 1 - p
            pltpu.make_async_copy(hp_hbm.at[idx2.at[pl.ds(q * G, G)]],
                                  rows2.at[q], gsem.at[q]).wait()
            _proc(q)
        for k in range((PEND - G) // 16):
            pend_src[pl.ds(k * 16, 16)] = pend_src[pl.ds(G + k * 16, 16)]
            pend_dl[pl.ds(k * 16, 16)] = pend_dl[pl.ds(G + k * 16, 16)]
        return pos_vec - G, fcnt + 1

    def _group(cb, g, pos_vec):
        d = dst_buf[pl.ds(cb * CH + g * 16, 16)]
        s = src_buf[pl.ds(cb * CH + g * 16, 16)]
        m = (d >= lo_vec) & (d < hi_vec)
        mi = jnp.where(m, zero16 + 1, zero16)
        pc = plsc.cumsum(mi)
        tgt = (pc - mi) + pos_vec
        plsc.store_scatter(pend_dl, [tgt], d - lo_vec, mask=m)
        plsc.store_scatter(pend_src, [tgt], s, mask=m)
        cnt = plsc.all_reduce_population_count(m)
        return pos_vec + cnt

    def _make_block(cb):
        def _block(b, carry):
            pos_vec, fcnt = carry
            for k in range(UNROLL):
                pos_vec = _group(cb, b * UNROLL + k, pos_vec)
            return lax.cond(pos_vec[0] >= G, _fire, lambda a: a,
                            (pos_vec, fcnt))
        return _block

    def _start_chunk(c):
        cb = lax.rem(c, 2)
        pltpu.async_copy(dst_hbm.at[pl.ds(c * CH, CH)],
                         dst_buf.at[pl.ds(cb * CH, CH)], dsem.at[cb])
        pltpu.async_copy(src_hbm.at[pl.ds(c * CH, CH)],
                         src_buf.at[pl.ds(cb * CH, CH)], ssem.at[cb])

    def _chunk(c, carry):
        cb = lax.rem(c, 2)
        pltpu.make_async_copy(dst_hbm.at[pl.ds(c * CH, CH)],
                              dst_buf.at[pl.ds(cb * CH, CH)],
                              dsem.at[cb]).wait()
        pltpu.make_async_copy(src_hbm.at[pl.ds(c * CH, CH)],
                              src_buf.at[pl.ds(cb * CH, CH)],
                              ssem.at[cb]).wait()

        @pl.when(c + 1 < NCHUNK)
        def _():
            _start_chunk(c + 1)

        return lax.fori_loop(0, NBLK, _make_block(cb), carry)

    _start_chunk(0)
    pos_vec, fcnt = lax.fori_loop(0, NCHUNK, _chunk, (zero16, 0))
    pos_vec, fcnt = _fire((pos_vec, fcnt))
    p_last = lax.rem(fcnt - 1, 2)
    pltpu.make_async_copy(hp_hbm.at[idx2.at[pl.ds(p_last * G, G)]],
                          rows2.at[p_last], gsem.at[p_last]).wait()
    _proc(p_last)

    def _merge_row(i, carry):
        for j in range(D // 32):
            a = plsc.bitcast(acc0[i, pl.ds(j * 16, 16)], jnp.bfloat16)
            b = plsc.bitcast(acc1[i, pl.ds(j * 16, 16)], jnp.bfloat16)
            acc0[i, pl.ds(j * 16, 16)] = plsc.bitcast(jnp.maximum(a, b),
                                                      jnp.int32)
        return carry

    lax.fori_loop(0, RANGE, _merge_row, 0)
    pltpu.sync_copy(acc0.at[pl.ds(0, RANGE)], out_hbm.at[pl.ds(lo, RANGE)])


def kernel(x, edge_index, Wp1, bp1, Ws1, Wn1, b1, Wp2, bp2, Ws2, Wn2, b2):
    src = edge_index[0]
    dst = edge_index[1]
    hp1, xs1 = _mm1(x, Wp1, bp1.reshape(1, D), Ws1)
    agg1 = _segmax(hp1, src, dst)
    hp2, hs2 = _mm2(xs1, agg1, Wn1, b1.reshape(1, D), Wp2, bp2.reshape(1, D),
                    Ws2)
    agg2 = _segmax(hp2, src, dst)
    out = _mm3(hs2, agg2, Wn2, b2.reshape(1, D))
    return out
